# Initial kernel scaffold; baseline (speedup 1.0000x reference)
import functools

import jax
import jax.numpy as jnp
from jax import lax
from jax.experimental import pallas as pl
from jax.experimental.pallas import tpu as pltpu
from jax.experimental.pallas import tpu_sc as plsc

_N = 4096
_D = 128
_C = 64
_L = 16
_W = 16
_B = 128
_NB = _N // (_W * _B)
_NCH = _D // _L


def _rsqrt16(x):
    x = jnp.maximum(x, jnp.full((_L,), 1e-35, jnp.float32))
    i = plsc.bitcast(x, jnp.int32)
    i = jnp.full((_L,), 0x5F3759DF, jnp.int32) - lax.shift_right_logical(
        i, jnp.full((_L,), 1, jnp.int32)
    )
    y = plsc.bitcast(i, jnp.float32)
    xh = x * jnp.float32(0.5)
    for _ in range(3):
        y = y * (jnp.float32(1.5) - xh * y * y)
    return y


def _body(ps_hbm, zs_hbm, tg_hbm, out_hbm, pv, zv, tv, ones_v, zb, zcb, dbuf,
          obuf, lp, lz, lc, ld, shp, shz, shc, shd):
    w = lax.axis_index("s")
    zero16 = jnp.zeros((_L,), jnp.float32)
    one16 = jnp.ones((_L,), jnp.float32)

    for r in range(4):
        for c in range(_NCH):
            zb[r, pl.ds(c * _L, _L)] = zero16
        zcb[r, :] = zero16

    def _ones_row(r, carry):
        ones_v[r, :] = one16
        return carry

    lax.fori_loop(0, _B, _ones_row, 0)

    pltpu.sync_copy(zb, shp.at[pl.ds(w * 4, 4)])
    pltpu.sync_copy(zb, shz.at[pl.ds(w * 4, 4)])
    pltpu.sync_copy(zcb, shc.at[pl.ds(w * 4, 4)])
    plsc.subcore_barrier()

    pltpu.sync_copy(ps_hbm.at[pl.ds(w * _NB, _NB)], pv)
    pltpu.sync_copy(zs_hbm.at[pl.ds(w * _NB, _NB)], zv)
    pltpu.sync_copy(tg_hbm.at[pl.ds(w * _NB, _NB)], tv)

    dacc = zero16
    for j in range(_NB):

        def _row(r, dacc, j=j):
            pc = [pv[j, r, pl.ds(c * _L, _L)] for c in range(_NCH)]
            zc = [zv[j, r, pl.ds(c * _L, _L)] for c in range(_NCH)]
            ssp = pc[0] * pc[0]
            ssz = zc[0] * zc[0]
            for c in range(1, _NCH):
                ssp += pc[c] * pc[c]
                ssz += zc[c] * zc[c]
            sp = jnp.full((_L,), jnp.sum(ssp))
            sz = jnp.full((_L,), jnp.sum(ssz))
            ip = jnp.minimum(_rsqrt16(sp), jnp.full((_L,), 1e8, jnp.float32))
            iz = jnp.minimum(_rsqrt16(sz), jnp.full((_L,), 1e8, jnp.float32))
            for c in range(_NCH):
                pn = pc[c] * ip
                zn = zc[c] * iz
                pv[j, r, pl.ds(c * _L, _L)] = pn
                zv[j, r, pl.ds(c * _L, _L)] = zn
                dacc = dacc + pn * zn
            return dacc

        dacc = lax.fori_loop(0, _B, _row, dacc)

    for j in range(_NB):
        pltpu.sync_copy(pv.at[j], shp.at[tv.at[j]], add=True)
        pltpu.sync_copy(zv.at[j], shz.at[tv.at[j]], add=True)
        pltpu.sync_copy(ones_v, shc.at[tv.at[j]], add=True)
    dbuf[:] = dacc
    pltpu.sync_copy(dbuf, shd.at[w])
    plsc.subcore_barrier()

    @pl.when(w == 0)
    def _final():
        pltpu.sync_copy(shp, lp)
        pltpu.sync_copy(shz, lz)
        pltpu.sync_copy(shc, lc)
        pltpu.sync_copy(shd, ld)

        def _cls(c, carry):
            sacc, nacc = carry
            for k in range(_NCH):
                sacc = sacc + lp[c, pl.ds(k * _L, _L)] * lz[c, pl.ds(k * _L, _L)]
            v = lc[c, :]
            return sacc, nacc + v * v

        sacc, nacc = lax.fori_loop(0, _C, _cls, (zero16, zero16))
        dtot = zero16
        for ww in range(_W):
            dtot = dtot + ld[ww, :]
        s_all = jnp.sum(sacc)
        d_all = jnp.sum(dtot)
        pairs = jnp.max(nacc) - jnp.float32(_N)
        loss = -(s_all - d_all) / pairs
        obuf[:] = jnp.full((_L,), loss)
        pltpu.sync_copy(obuf, out_hbm)


_mesh = plsc.VectorSubcoreMesh(
    core_axis_name="c", subcore_axis_name="s", num_cores=1
)

_simsiam_sc = functools.partial(
    pl.kernel,
    out_type=jax.ShapeDtypeStruct((_L,), jnp.float32),
    mesh=_mesh,
    scratch_types=[
        pltpu.VMEM((_NB, _B, _D), jnp.float32),
        pltpu.VMEM((_NB, _B, _D), jnp.float32),
        pltpu.VMEM((_NB, _B), jnp.int32),
        pltpu.VMEM((_B, _L), jnp.float32),
        pltpu.VMEM((4, _D), jnp.float32),
        pltpu.VMEM((4, _L), jnp.float32),
        pltpu.VMEM((_L,), jnp.float32),
        pltpu.VMEM((_L,), jnp.float32),
        pltpu.VMEM((_C, _D), jnp.float32),
        pltpu.VMEM((_C, _D), jnp.float32),
        pltpu.VMEM((_C, _L), jnp.float32),
        pltpu.VMEM((_W, _L), jnp.float32),
        pltpu.VMEM_SHARED((_C, _D), jnp.float32),
        pltpu.VMEM_SHARED((_C, _D), jnp.float32),
        pltpu.VMEM_SHARED((_C, _L), jnp.float32),
        pltpu.VMEM_SHARED((_W, _L), jnp.float32),
    ],
)(_body)


def kernel(ps, zs, targets):
    ps_r = ps.reshape(_W * _NB, _B, _D)
    zs_r = zs.reshape(_W * _NB, _B, _D)
    tg_r = targets.reshape(_W * _NB, _B).astype(jnp.int32)
    out = _simsiam_sc(ps_r, zs_r, tg_r)
    return out[0]


# speedup vs baseline: 4.3147x; 4.3147x over previous
import functools

import jax
import jax.numpy as jnp
from jax import lax
from jax.experimental import pallas as pl
from jax.experimental.pallas import tpu as pltpu
from jax.experimental.pallas import tpu_sc as plsc

_N = 4096
_D = 128
_C = 64
_L = 16
_W = 16
_B = 128
_NB = _N // (_W * _B)
_NCH = _D // _L


def _rsqrt16(x):
    x = jnp.maximum(x, jnp.full((_L,), 1e-35, jnp.float32))
    i = plsc.bitcast(x, jnp.int32)
    i = jnp.full((_L,), 0x5F3759DF, jnp.int32) - lax.shift_right_logical(
        i, jnp.full((_L,), 1, jnp.int32)
    )
    y = plsc.bitcast(i, jnp.float32)
    xh = x * jnp.float32(0.5)
    for _ in range(3):
        y = y * (jnp.float32(1.5) - xh * y * y)
    return y


def _body(ps_hbm, zs_hbm, tg_hbm, out_hbm, pv, zv, tv, ones_v, zb, zcb, dbuf,
          obuf, lp, lz, lm, shp, shz, shm):
    w = lax.axis_index("s")
    zero16 = jnp.zeros((_L,), jnp.float32)
    one16 = jnp.ones((_L,), jnp.float32)

    for r in range(4):
        for c in range(_NCH):
            zb[r, pl.ds(c * _L, _L)] = zero16
    for r in range(5):
        for c in range(_NCH):
            zcb[r, pl.ds(c * _L, _L)] = zero16
    for c in range(_NCH):
        dbuf[pl.ds(c * _L, _L)] = zero16

    def _ones_row(r, carry):
        for c in range(_NCH):
            ones_v[r, pl.ds(c * _L, _L)] = one16
        return carry

    lax.fori_loop(0, _B, _ones_row, 0)

    pltpu.sync_copy(zb, shp.at[pl.ds(w * 4, 4)])
    pltpu.sync_copy(zb, shz.at[pl.ds(w * 4, 4)])
    pltpu.sync_copy(zcb, shm.at[pl.ds(w * 5, 5)])
    plsc.subcore_barrier()

    pltpu.sync_copy(ps_hbm.at[pl.ds(w * _NB, _NB)], pv)
    pltpu.sync_copy(zs_hbm.at[pl.ds(w * _NB, _NB)], zv)
    pltpu.sync_copy(tg_hbm.at[pl.ds(w * _NB, _NB)], tv)

    dacc = zero16
    for j in range(_NB):

        def _row(r, dacc, j=j):
            pc = [pv[j, r, pl.ds(c * _L, _L)] for c in range(_NCH)]
            zc = [zv[j, r, pl.ds(c * _L, _L)] for c in range(_NCH)]
            ssp = pc[0] * pc[0]
            ssz = zc[0] * zc[0]
            for c in range(1, _NCH):
                ssp += pc[c] * pc[c]
                ssz += zc[c] * zc[c]
            sp = jnp.full((_L,), jnp.sum(ssp))
            sz = jnp.full((_L,), jnp.sum(ssz))
            ip = jnp.minimum(_rsqrt16(sp), jnp.full((_L,), 1e8, jnp.float32))
            iz = jnp.minimum(_rsqrt16(sz), jnp.full((_L,), 1e8, jnp.float32))
            for c in range(_NCH):
                pn = pc[c] * ip
                zn = zc[c] * iz
                pv[j, r, pl.ds(c * _L, _L)] = pn
                zv[j, r, pl.ds(c * _L, _L)] = zn
                dacc = dacc + pn * zn
            return dacc

        dacc = lax.fori_loop(0, _B, _row, dacc)

    for j in range(_NB):
        pltpu.sync_copy(pv.at[j], shp.at[tv.at[j]], add=True)
        pltpu.sync_copy(zv.at[j], shz.at[tv.at[j]], add=True)
        pltpu.sync_copy(ones_v, shm.at[tv.at[j]], add=True)
    dbuf[pl.ds(0, _L)] = dacc
    pltpu.sync_copy(dbuf, shm.at[64 + w])
    plsc.subcore_barrier()

    @pl.when(w == 0)
    def _final():
        pltpu.sync_copy(shp, lp)
        pltpu.sync_copy(shz, lz)
        pltpu.sync_copy(shm, lm)

        def _cls(c, carry):
            sacc, nacc = carry
            for k in range(_NCH):
                sacc = sacc + lp[c, pl.ds(k * _L, _L)] * lz[c, pl.ds(k * _L, _L)]
            v = lm[c, pl.ds(0, _L)]
            return sacc, nacc + v * v

        sacc, nacc = lax.fori_loop(0, _C, _cls, (zero16, zero16))
        dtot = zero16
        for ww in range(_W):
            dtot = dtot + lm[64 + ww, pl.ds(0, _L)]
        s_vec = jnp.full((_L,), jnp.sum(sacc))
        d_vec = jnp.full((_L,), jnp.sum(dtot))
        pairs_vec = nacc - jnp.float32(_N)
        obuf[:] = -(s_vec - d_vec) / pairs_vec
        pltpu.sync_copy(obuf, out_hbm)


_mesh = plsc.VectorSubcoreMesh(
    core_axis_name="c", subcore_axis_name="s", num_cores=1, num_subcores=_W
)

_simsiam_sc = functools.partial(
    pl.kernel,
    out_type=jax.ShapeDtypeStruct((_L,), jnp.float32),
    mesh=_mesh,
    compiler_params=pltpu.CompilerParams(needs_layout_passes=False),
    scratch_types=[
        pltpu.VMEM((_NB, _B, _D), jnp.float32),
        pltpu.VMEM((_NB, _B, _D), jnp.float32),
        pltpu.VMEM((_NB, _B), jnp.int32),
        pltpu.VMEM((_B, _D), jnp.float32),
        pltpu.VMEM((4, _D), jnp.float32),
        pltpu.VMEM((5, _D), jnp.float32),
        pltpu.VMEM((_D,), jnp.float32),
        pltpu.VMEM((_L,), jnp.float32),
        pltpu.VMEM((_C, _D), jnp.float32),
        pltpu.VMEM((_C, _D), jnp.float32),
        pltpu.VMEM((_C + _W, _D), jnp.float32),
        pltpu.VMEM_SHARED((_C, _D), jnp.float32),
        pltpu.VMEM_SHARED((_C, _D), jnp.float32),
        pltpu.VMEM_SHARED((_C + _W, _D), jnp.float32),
    ],
)(_body)


def kernel(ps, zs, targets):
    ps_r = ps.reshape(_W * _NB, _B, _D)
    zs_r = zs.reshape(_W * _NB, _B, _D)
    tg_r = targets.reshape(_W * _NB, _B).astype(jnp.int32)
    out = _simsiam_sc(ps_r, zs_r, tg_r)
    return out[0]


# speedup vs baseline: 5.0799x; 1.1773x over previous
import functools

import jax
import jax.numpy as jnp
from jax import lax
from jax.experimental import pallas as pl
from jax.experimental.pallas import tpu as pltpu
from jax.experimental.pallas import tpu_sc as plsc

_N = 4096
_D = 128
_C = 64
_L = 16
_W = 16
_B = 128
_NB = _N // (_W * _B)
_NCH = _D // _L


def _rsqrt16(x):
    x = jnp.maximum(x, jnp.full((_L,), 1e-35, jnp.float32))
    i = plsc.bitcast(x, jnp.int32)
    i = jnp.full((_L,), 0x5F3759DF, jnp.int32) - lax.shift_right_logical(
        i, jnp.full((_L,), 1, jnp.int32)
    )
    y = plsc.bitcast(i, jnp.float32)
    xh = x * jnp.float32(0.5)
    for _ in range(2):
        y = y * (jnp.float32(1.5) - xh * y * y)
    return y


def _lanesum(v):
    for s in (1, 2, 4, 8):
        idx = lax.iota(jnp.int32, _L) ^ s
        v = v + jnp.take_along_axis(v, idx, axis=0)
    return v


def _body(ps_hbm, zs_hbm, tg_hbm, out_hbm, pv, zv, tv, ones_v, zb, zcb, dbuf,
          obuf, lp, lz, lm, shp, shz, shm, dma_sem):
    w = lax.axis_index("s")
    zero16 = jnp.zeros((_L,), jnp.float32)
    one16 = jnp.ones((_L,), jnp.float32)

    cp_p = pltpu.async_copy(ps_hbm.at[pl.ds(w * _NB, _NB)], pv, dma_sem)
    cp_z = pltpu.async_copy(zs_hbm.at[pl.ds(w * _NB, _NB)], zv, dma_sem)
    cp_t = pltpu.async_copy(tg_hbm.at[pl.ds(w * _NB, _NB)], tv, dma_sem)

    for r in range(4):
        for c in range(_NCH):
            zb[r, pl.ds(c * _L, _L)] = zero16
    for r in range(5):
        for c in range(_NCH):
            zcb[r, pl.ds(c * _L, _L)] = zero16
    for c in range(_NCH):
        dbuf[pl.ds(c * _L, _L)] = zero16

    def _ones_row(r, carry):
        for c in range(_NCH):
            ones_v[r, pl.ds(c * _L, _L)] = one16
        return carry

    lax.fori_loop(0, _B, _ones_row, 0)

    pltpu.sync_copy(zb, shp.at[pl.ds(w * 4, 4)])
    pltpu.sync_copy(zb, shz.at[pl.ds(w * 4, 4)])
    pltpu.sync_copy(zcb, shm.at[pl.ds(w * 5, 5)])
    plsc.subcore_barrier()

    cp_p.wait()
    cp_z.wait()
    cp_t.wait()

    dacc = zero16
    for j in range(_NB):

        @plsc.parallel_loop(0, _B, carry=dacc)
        def dacc(r, d, j=j):
            pc = [pv[j, r, pl.ds(c * _L, _L)] for c in range(_NCH)]
            zc = [zv[j, r, pl.ds(c * _L, _L)] for c in range(_NCH)]
            ssp = pc[0] * pc[0]
            ssz = zc[0] * zc[0]
            for c in range(1, _NCH):
                ssp += pc[c] * pc[c]
                ssz += zc[c] * zc[c]
            sp = _lanesum(ssp)
            sz = _lanesum(ssz)
            ip = jnp.minimum(_rsqrt16(sp), jnp.full((_L,), 1e8, jnp.float32))
            iz = jnp.minimum(_rsqrt16(sz), jnp.full((_L,), 1e8, jnp.float32))
            for c in range(_NCH):
                pn = pc[c] * ip
                zn = zc[c] * iz
                pv[j, r, pl.ds(c * _L, _L)] = pn
                zv[j, r, pl.ds(c * _L, _L)] = zn
                d = d + pn * zn
            return d

    for j in range(_NB):
        pltpu.sync_copy(pv.at[j], shp.at[tv.at[j]], add=True)
        pltpu.sync_copy(zv.at[j], shz.at[tv.at[j]], add=True)
        pltpu.sync_copy(ones_v, shm.at[tv.at[j]], add=True)
    dbuf[pl.ds(0, _L)] = dacc
    pltpu.sync_copy(dbuf, shm.at[64 + w])
    plsc.subcore_barrier()

    @pl.when(w == 0)
    def _final():
        pltpu.sync_copy(shp, lp)
        pltpu.sync_copy(shz, lz)
        pltpu.sync_copy(shm, lm)

        def _cls(c, carry):
            sacc, nacc = carry
            for k in range(_NCH):
                sacc = sacc + lp[c, pl.ds(k * _L, _L)] * lz[c, pl.ds(k * _L, _L)]
            v = lm[c, pl.ds(0, _L)]
            return sacc, nacc + v * v

        sacc, nacc = lax.fori_loop(0, _C, _cls, (zero16, zero16))
        dtot = zero16
        for ww in range(_W):
            dtot = dtot + lm[64 + ww, pl.ds(0, _L)]
        s_vec = _lanesum(sacc)
        d_vec = _lanesum(dtot)
        pairs_vec = nacc - jnp.float32(_N)
        obuf[:] = -(s_vec - d_vec) / pairs_vec
        pltpu.sync_copy(obuf, out_hbm)


_mesh = plsc.VectorSubcoreMesh(
    core_axis_name="c", subcore_axis_name="s", num_cores=1, num_subcores=_W
)

_simsiam_sc = functools.partial(
    pl.kernel,
    out_type=jax.ShapeDtypeStruct((_L,), jnp.float32),
    mesh=_mesh,
    compiler_params=pltpu.CompilerParams(needs_layout_passes=False),
    scratch_types=[
        pltpu.VMEM((_NB, _B, _D), jnp.float32),
        pltpu.VMEM((_NB, _B, _D), jnp.float32),
        pltpu.VMEM((_NB, _B), jnp.int32),
        pltpu.VMEM((_B, _D), jnp.float32),
        pltpu.VMEM((4, _D), jnp.float32),
        pltpu.VMEM((5, _D), jnp.float32),
        pltpu.VMEM((_D,), jnp.float32),
        pltpu.VMEM((_L,), jnp.float32),
        pltpu.VMEM((_C, _D), jnp.float32),
        pltpu.VMEM((_C, _D), jnp.float32),
        pltpu.VMEM((_C + _W, _D), jnp.float32),
        pltpu.VMEM_SHARED((_C, _D), jnp.float32),
        pltpu.VMEM_SHARED((_C, _D), jnp.float32),
        pltpu.VMEM_SHARED((_C + _W, _D), jnp.float32),
        pltpu.SemaphoreType.DMA,
    ],
)(_body)


def kernel(ps, zs, targets):
    ps_r = ps.reshape(_W * _NB, _B, _D)
    zs_r = zs.reshape(_W * _NB, _B, _D)
    tg_r = targets.reshape(_W * _NB, _B).astype(jnp.int32)
    out = _simsiam_sc(ps_r, zs_r, tg_r)
    return out[0]


# speedup vs baseline: 5.1584x; 1.0154x over previous
import functools

import jax
import jax.numpy as jnp
from jax import lax
from jax.experimental import pallas as pl
from jax.experimental.pallas import tpu as pltpu
from jax.experimental.pallas import tpu_sc as plsc

_N = 4096
_D = 128
_C = 64
_L = 16
_W = 16
_B = 128
_NB = _N // (_W * _B)
_NCH = _D // _L


def _rsqrt16(x):
    x = jnp.maximum(x, jnp.full((_L,), 1e-35, jnp.float32))
    i = plsc.bitcast(x, jnp.int32)
    i = jnp.full((_L,), 0x5F3759DF, jnp.int32) - lax.shift_right_logical(
        i, jnp.full((_L,), 1, jnp.int32)
    )
    y = plsc.bitcast(i, jnp.float32)
    xh = x * jnp.float32(0.5)
    for _ in range(2):
        y = y * (jnp.float32(1.5) - xh * y * y)
    return y


def _lanesum(v):
    for s in (1, 2, 4, 8):
        idx = lax.iota(jnp.int32, _L) ^ s
        v = v + jnp.take_along_axis(v, idx, axis=0)
    return v


def _body(ps_hbm, zs_hbm, tg_hbm, out_hbm, pv, zv, tv, ones_v, zb, zcb, dbuf,
          obuf, lp, lz, lm, shp, shz, shm, dma_sem):
    w = lax.axis_index("s")
    zero16 = jnp.zeros((_L,), jnp.float32)
    one16 = jnp.ones((_L,), jnp.float32)

    cp_p = pltpu.async_copy(ps_hbm.at[pl.ds(w * _NB, _NB)], pv, dma_sem)
    cp_z = pltpu.async_copy(zs_hbm.at[pl.ds(w * _NB, _NB)], zv, dma_sem)
    cp_t = pltpu.async_copy(tg_hbm.at[pl.ds(w * _NB, _NB)], tv, dma_sem)

    for r in range(4):
        for c in range(_NCH):
            zb[r, pl.ds(c * _L, _L)] = zero16
    for r in range(5):
        for c in range(_NCH):
            zcb[r, pl.ds(c * _L, _L)] = zero16
    for c in range(_NCH):
        dbuf[pl.ds(c * _L, _L)] = zero16

    def _ones_row(r, carry):
        for c in range(_NCH):
            ones_v[r, pl.ds(c * _L, _L)] = one16
        return carry

    lax.fori_loop(0, _B, _ones_row, 0)

    pltpu.sync_copy(zb, shp.at[pl.ds(w * 4, 4)])
    pltpu.sync_copy(zb, shz.at[pl.ds(w * 4, 4)])
    pltpu.sync_copy(zcb, shm.at[pl.ds(w * 5, 5)])
    plsc.subcore_barrier()

    cp_p.wait()
    cp_z.wait()
    cp_t.wait()

    daccs = (zero16,) * _NCH
    for j in range(_NB):

        @plsc.parallel_loop(0, _B, carry=daccs)
        def daccs(r, ds_c, j=j):
            pc = [pv[j, r, pl.ds(c * _L, _L)] for c in range(_NCH)]
            zc = [zv[j, r, pl.ds(c * _L, _L)] for c in range(_NCH)]
            ssp = pc[0] * pc[0]
            ssz = zc[0] * zc[0]
            for c in range(1, _NCH):
                ssp += pc[c] * pc[c]
                ssz += zc[c] * zc[c]
            sp = _lanesum(ssp)
            sz = _lanesum(ssz)
            ip = jnp.minimum(_rsqrt16(sp), jnp.full((_L,), 1e8, jnp.float32))
            iz = jnp.minimum(_rsqrt16(sz), jnp.full((_L,), 1e8, jnp.float32))
            out = []
            for c in range(_NCH):
                pn = pc[c] * ip
                zn = zc[c] * iz
                pv[j, r, pl.ds(c * _L, _L)] = pn
                zv[j, r, pl.ds(c * _L, _L)] = zn
                out.append(ds_c[c] + pn * zn)
            return tuple(out)

    dacc = daccs[0]
    for c in range(1, _NCH):
        dacc = dacc + daccs[c]

    for j in range(_NB):
        pltpu.sync_copy(pv.at[j], shp.at[tv.at[j]], add=True)
        pltpu.sync_copy(zv.at[j], shz.at[tv.at[j]], add=True)
        pltpu.sync_copy(ones_v, shm.at[tv.at[j]], add=True)
    dbuf[pl.ds(0, _L)] = dacc
    pltpu.sync_copy(dbuf, shm.at[64 + w])
    plsc.subcore_barrier()

    @pl.when(w == 0)
    def _final():
        pltpu.sync_copy(shp, lp)
        pltpu.sync_copy(shz, lz)
        pltpu.sync_copy(shm, lm)

        def _cls(c, carry):
            sacc, nacc = carry
            for k in range(_NCH):
                sacc = sacc + lp[c, pl.ds(k * _L, _L)] * lz[c, pl.ds(k * _L, _L)]
            v = lm[c, pl.ds(0, _L)]
            return sacc, nacc + v * v

        sacc, nacc = lax.fori_loop(0, _C, _cls, (zero16, zero16))
        dtot = zero16
        for ww in range(_W):
            dtot = dtot + lm[64 + ww, pl.ds(0, _L)]
        s_vec = _lanesum(sacc)
        d_vec = _lanesum(dtot)
        pairs_vec = nacc - jnp.float32(_N)
        obuf[:] = -(s_vec - d_vec) / pairs_vec
        pltpu.sync_copy(obuf, out_hbm)


_mesh = plsc.VectorSubcoreMesh(
    core_axis_name="c", subcore_axis_name="s", num_cores=1, num_subcores=_W
)

_simsiam_sc = functools.partial(
    pl.kernel,
    out_type=jax.ShapeDtypeStruct((_L,), jnp.float32),
    mesh=_mesh,
    compiler_params=pltpu.CompilerParams(needs_layout_passes=False),
    scratch_types=[
        pltpu.VMEM((_NB, _B, _D), jnp.float32),
        pltpu.VMEM((_NB, _B, _D), jnp.float32),
        pltpu.VMEM((_NB, _B), jnp.int32),
        pltpu.VMEM((_B, _D), jnp.float32),
        pltpu.VMEM((4, _D), jnp.float32),
        pltpu.VMEM((5, _D), jnp.float32),
        pltpu.VMEM((_D,), jnp.float32),
        pltpu.VMEM((_L,), jnp.float32),
        pltpu.VMEM((_C, _D), jnp.float32),
        pltpu.VMEM((_C, _D), jnp.float32),
        pltpu.VMEM((_C + _W, _D), jnp.float32),
        pltpu.VMEM_SHARED((_C, _D), jnp.float32),
        pltpu.VMEM_SHARED((_C, _D), jnp.float32),
        pltpu.VMEM_SHARED((_C + _W, _D), jnp.float32),
        pltpu.SemaphoreType.DMA,
    ],
)(_body)


def kernel(ps, zs, targets):
    ps_r = ps.reshape(_W * _NB, _B, _D)
    zs_r = zs.reshape(_W * _NB, _B, _D)
    tg_r = targets.reshape(_W * _NB, _B).astype(jnp.int32)
    out = _simsiam_sc(ps_r, zs_r, tg_r)
    return out[0]


# speedup vs baseline: 5.7577x; 1.1162x over previous
import functools

import jax
import jax.numpy as jnp
from jax import lax
from jax.experimental import pallas as pl
from jax.experimental.pallas import tpu as pltpu
from jax.experimental.pallas import tpu_sc as plsc

_N = 4096
_D = 128
_C = 64
_L = 16
_NC = 2
_W = 16
_B = 128
_NCH = _D // _L
_TR = _C + _C + _C + _W


def _rsqrt16(x):
    x = jnp.maximum(x, jnp.full((_L,), 1e-35, jnp.float32))
    i = jnp.full((_L,), 0x5F3759DF, jnp.int32) - lax.shift_right_logical(
        plsc.bitcast(x, jnp.int32), jnp.full((_L,), 1, jnp.int32)
    )
    y = plsc.bitcast(i, jnp.float32)
    xh = x * jnp.float32(0.5)
    for _ in range(2):
        y = y * (jnp.float32(1.5) - xh * y * y)
    return y


def _lanesum(v):
    for s in (1, 2, 4, 8):
        idx = lax.iota(jnp.int32, _L) ^ s
        v = v + jnp.take_along_axis(v, idx, axis=0)
    return v


def _sc_body(ps_hbm, zs_hbm, tg_hbm, out_hbm, pv, zv, tv, ones_v, zb, zcb,
             dbuf, shp, shz, shm, dma_sem):
    ci = lax.axis_index("c")
    w = lax.axis_index("s")
    g = ci * _W + w
    zero16 = jnp.zeros((_L,), jnp.float32)
    one16 = jnp.ones((_L,), jnp.float32)

    cp_p = pltpu.async_copy(ps_hbm.at[g], pv, dma_sem)
    cp_z = pltpu.async_copy(zs_hbm.at[g], zv, dma_sem)
    cp_t = pltpu.async_copy(tg_hbm.at[g], tv, dma_sem)

    for r in range(4):
        for c in range(_NCH):
            zb[r, pl.ds(c * _L, _L)] = zero16
    for r in range(5):
        for c in range(_NCH):
            zcb[r, pl.ds(c * _L, _L)] = zero16
    for c in range(_NCH):
        dbuf[pl.ds(c * _L, _L)] = zero16

    @plsc.parallel_loop(0, _B)
    def _ones_row(r):
        for c in range(_NCH):
            ones_v[r, pl.ds(c * _L, _L)] = one16

    pltpu.sync_copy(zb, shp.at[pl.ds(w * 4, 4)])
    pltpu.sync_copy(zb, shz.at[pl.ds(w * 4, 4)])
    pltpu.sync_copy(zcb, shm.at[pl.ds(w * 5, 5)])
    plsc.subcore_barrier()

    cp_p.wait()
    cp_z.wait()
    cp_t.wait()

    @plsc.parallel_loop(0, _B, carry=(zero16,) * _NCH)
    def daccs(r, ds_c):
        pc = [pv[r, pl.ds(c * _L, _L)] for c in range(_NCH)]
        zc = [zv[r, pl.ds(c * _L, _L)] for c in range(_NCH)]
        ssp = pc[0] * pc[0]
        ssz = zc[0] * zc[0]
        for c in range(1, _NCH):
            ssp += pc[c] * pc[c]
            ssz += zc[c] * zc[c]
        sp = _lanesum(ssp)
        sz = _lanesum(ssz)
        ip = jnp.minimum(_rsqrt16(sp), jnp.full((_L,), 1e8, jnp.float32))
        iz = jnp.minimum(_rsqrt16(sz), jnp.full((_L,), 1e8, jnp.float32))
        out = []
        for c in range(_NCH):
            pn = pc[c] * ip
            zn = zc[c] * iz
            pv[r, pl.ds(c * _L, _L)] = pn
            zv[r, pl.ds(c * _L, _L)] = zn
            out.append(ds_c[c] + pn * zn)
        return tuple(out)

    dacc = daccs[0]
    for c in range(1, _NCH):
        dacc = dacc + daccs[c]

    pltpu.sync_copy(pv, shp.at[tv], add=True)
    pltpu.sync_copy(zv, shz.at[tv], add=True)
    pltpu.sync_copy(ones_v, shm.at[tv], add=True)
    dbuf[pl.ds(0, _L)] = dacc
    pltpu.sync_copy(dbuf, shm.at[_C + w])
    plsc.subcore_barrier()

    @pl.when(w == 0)
    def _dump():
        pltpu.sync_copy(shp, out_hbm.at[ci, pl.ds(0, _C)])
        pltpu.sync_copy(shz, out_hbm.at[ci, pl.ds(_C, _C)])
        pltpu.sync_copy(shm, out_hbm.at[ci, pl.ds(2 * _C, _C + _W)])


_mesh = plsc.VectorSubcoreMesh(
    core_axis_name="c", subcore_axis_name="s", num_cores=_NC, num_subcores=_W
)

_simsiam_sc = functools.partial(
    pl.kernel,
    out_type=jax.ShapeDtypeStruct((_NC, _TR, _D), jnp.float32),
    mesh=_mesh,
    compiler_params=pltpu.CompilerParams(needs_layout_passes=False),
    scratch_types=[
        pltpu.VMEM((_B, _D), jnp.float32),
        pltpu.VMEM((_B, _D), jnp.float32),
        pltpu.VMEM((_B,), jnp.int32),
        pltpu.VMEM((_B, _D), jnp.float32),
        pltpu.VMEM((4, _D), jnp.float32),
        pltpu.VMEM((5, _D), jnp.float32),
        pltpu.VMEM((_D,), jnp.float32),
        pltpu.VMEM_SHARED((_C, _D), jnp.float32),
        pltpu.VMEM_SHARED((_C, _D), jnp.float32),
        pltpu.VMEM_SHARED((_C + _W, _D), jnp.float32),
        pltpu.SemaphoreType.DMA,
    ],
)(_sc_body)


def _tc_combine_body(t_ref, out_ref):
    sp = t_ref[0, 0:_C, :] + t_ref[1, 0:_C, :]
    sz = t_ref[0, _C:2 * _C, :] + t_ref[1, _C:2 * _C, :]
    cnt = t_ref[0, 2 * _C:3 * _C, :] + t_ref[1, 2 * _C:3 * _C, :]
    dg = t_ref[0, 3 * _C:, :] + t_ref[1, 3 * _C:, :]
    s_all = jnp.sum(sp * sz)
    d_all = jnp.sum(dg)
    nc = cnt[:, 0:1]
    pairs = jnp.sum(nc * nc) - jnp.float32(_N)
    out_ref[0, 0] = -(s_all - d_all) / pairs


_tc_combine = pl.pallas_call(
    _tc_combine_body,
    out_shape=jax.ShapeDtypeStruct((1, 1), jnp.float32),
    out_specs=pl.BlockSpec(memory_space=pltpu.MemorySpace.SMEM),
)


def kernel(ps, zs, targets):
    ps_r = ps.reshape(_NC * _W, _B, _D)
    zs_r = zs.reshape(_NC * _W, _B, _D)
    tg_r = targets.reshape(_NC * _W, _B).astype(jnp.int32)
    tabs = _simsiam_sc(ps_r, zs_r, tg_r)
    return _tc_combine(tabs).reshape(())
